# Initial kernel scaffold; baseline (speedup 1.0000x reference)
import jax
import jax.numpy as jnp
from jax.experimental import pallas as pl


def kernel(risks):
    raise NotImplementedError("write your pallas kernel here")


# speedup vs baseline: 29.1746x; 29.1746x over previous
import jax
import jax.numpy as jnp
from jax import lax
from jax.experimental import pallas as pl
from jax.experimental.pallas import tpu as pltpu
from jax.experimental.pallas import tpu_sc as plsc

N = 1_000_000
K = 500_000
NSUB = 16
CHUNK = 62_496
TAIL = N - NSUB * CHUNK
DATA_WORDS = CHUNK + TAIL
ITERS = CHUNK // 16
ITERS_TAIL = DATA_WORDS // 16
B = 1024
BROWS = B // 16


def _lane0(x):
    return lax.squeeze(lax.slice(x, (0,), (1,)), (0,))


def _find_bucket(ref, lane, target):
    zeros = jnp.zeros((16,), jnp.float32)

    def body(i, carry):
        run, b, cab, cat = carry
        c = (BROWS - 1) - i
        v = ref[c]
        rv = lax.rev(v, (0,))
        rc = lax.cumsum(rv, axis=0)
        s = lax.squeeze(lax.slice(rc, (15,), (16,)), (0,))
        crosses = jnp.logical_and(run < target, run + s >= target)

        def hit_fn():
            hit = (run + rc) >= target
            mv = plsc.all_reduce_ffs(hit)
            sel = lane == mv
            m = _lane0(mv)
            nb = c * 16 + 15 - m
            ncab = run + jnp.sum(jnp.where(sel, rc - rv, zeros))
            ncat = jnp.sum(jnp.where(sel, rv, zeros))
            return (nb, ncab, ncat)

        b, cab, cat = lax.cond(crosses, hit_fn, lambda: (b, cab, cat))
        return (run + s, b, cab, cat)

    f0 = jnp.float32(0)
    _, b, cab, cat = lax.fori_loop(
        0, BROWS, body, (f0, jnp.int32(0), f0, f0))
    return b, cab, cat


def _sortable_key(x):
    u = lax.bitcast_convert_type(x, jnp.int32)
    return u ^ lax.shift_right_logical(lax.shift_right_arithmetic(u, 31), 1)


def _body(risks_hbm, out_hbm, data_v, histc_v, sum2_v, stage1_v, stage2_v,
          stageA_v, stageB_v, idx64_v, merged1_sh, merged2c_sh, merged2s_sh,
          sumab_sh, sem):
    wid = lax.axis_index("s")
    lane = lax.iota(jnp.int32, 16)
    zeros = jnp.zeros((16,), jnp.float32)
    ones = jnp.ones((16,), jnp.float32)
    laneoff = lane * B

    cp = pltpu.async_copy(risks_hbm.at[pl.ds(wid * CHUNK, CHUNK)],
                          data_v.at[pl.ds(0, CHUNK)], sem)

    @plsc.parallel_loop(0, B, unroll=8)
    def _(i):
        histc_v[pl.ds(i * 16, 16)] = zeros

    @plsc.parallel_loop(0, BROWS, unroll=4)
    def _(c):
        stage1_v[c] = zeros
    for c4 in range(BROWS // 16):
        idx64_v[pl.ds(c4 * 16, 16)] = lane + c4 * 16

    @pl.when(wid == 0)
    def _():
        pltpu.sync_copy(stage1_v, merged1_sh)
        pltpu.sync_copy(stage1_v, merged2c_sh)
        pltpu.sync_copy(stage1_v.at[pl.ds(0, 16)], sumab_sh)

    @pl.when(wid == NSUB - 1)
    def _():
        pltpu.sync_copy(risks_hbm.at[pl.ds(NSUB * CHUNK, TAIL)],
                        data_v.at[pl.ds(CHUNK, TAIL)])
    plsc.subcore_barrier()

    cp.wait()
    iters_me = jnp.where(wid == NSUB - 1, ITERS_TAIL, ITERS)

    laneoff512 = laneoff + 512

    @plsc.parallel_loop(0, iters_me, unroll=16)
    def _(i):
        x = data_v[pl.ds(i * 16, 16)]
        key = _sortable_key(x)
        idx = laneoff512 + lax.shift_right_arithmetic(key, 22)
        plsc.addupdate_scatter(histc_v, [idx], ones)

    @plsc.parallel_loop(0, BROWS, unroll=2)
    def _(c):
        acc_ = zeros
        for l in range(16):
            acc_ = acc_ + histc_v[pl.ds(l * B + c * 16, 16)]
        stage1_v[c] = acc_
    pltpu.sync_copy(stage1_v, merged1_sh.at[idx64_v], add=True)

    @plsc.parallel_loop(0, B, unroll=8)
    def _(i):
        histc_v[pl.ds(i * 16, 16)] = zeros
    plsc.subcore_barrier()

    pltpu.sync_copy(merged1_sh, stage1_v)
    kf = jnp.float32(K)
    b1, cab1, _ = _find_bucket(stage1_v, lane, kf)
    rem1 = kf - cab1

    b1s = b1 - 512

    @plsc.parallel_loop(0, iters_me, unroll=16, carry=zeros)
    def acc(i, a):
        x = data_v[pl.ds(i * 16, 16)]
        key = _sortable_key(x)
        b1v = lax.shift_right_arithmetic(key, 22)
        a = a + jnp.where(b1v > b1s, x, zeros)
        inb = b1v == b1s
        idx2 = laneoff + (lax.shift_right_arithmetic(key, 12) & 1023)
        plsc.addupdate_scatter(histc_v, [idx2], ones, mask=inb)
        return a

    @plsc.parallel_loop(0, BROWS, unroll=2)
    def _(c):
        acc1 = zeros
        for l in range(16):
            acc1 = acc1 + histc_v[pl.ds(l * B + c * 16, 16)]
        stage1_v[c] = acc1
    stageA_v[...] = acc
    pltpu.sync_copy(stage1_v, merged2c_sh.at[idx64_v], add=True)
    pltpu.sync_copy(stageA_v, sumab_sh.at[wid])
    plsc.subcore_barrier()

    @pl.when(wid == 0)
    def _():
        pltpu.sync_copy(merged2c_sh, stage1_v)
        pltpu.sync_copy(sumab_sh, stageB_v)
        b2, cab2, _ = _find_bucket(stage1_v, lane, rem1)
        rem2 = rem1 - cab2

        hi_key = lax.shift_left(b1s, 22)

        @plsc.parallel_loop(0, BROWS, unroll=4, carry=(zeros, zeros))
        def fin_acc(c, carry):
            sa_v, ed_v = carry
            vc = stage1_v[c]
            gidx = c * 16 + lane
            ke = hi_key + lax.shift_left(gidx, 12)
            ue = jnp.where(ke < 0, ke ^ jnp.int32(0x7FFFFFFF), ke)
            edge = lax.bitcast_convert_type(ue, jnp.float32)
            sa_v = sa_v + jnp.where(gidx > b2, vc * edge, zeros)
            ed_v = ed_v + jnp.where(gidx == b2, edge, zeros)
            return (sa_v, ed_v)
        s_above2 = jnp.sum(fin_acc[0])
        edge_at = jnp.sum(fin_acc[1])

        @plsc.parallel_loop(0, NSUB, unroll=4, carry=zeros)
        def ts_acc(t, a):
            return a + stageB_v[t]
        sum_above1 = jnp.sum(ts_acc)

        tail = (zeros + rem2) * (zeros + edge_at)
        stageA_v[...] = ((zeros + sum_above1 + s_above2) + tail) / (zeros + kf)
        pltpu.sync_copy(stageA_v, out_hbm)


_cvar_call = pl.kernel(
    _body,
    out_type=jax.ShapeDtypeStruct((16,), jnp.float32),
    mesh=plsc.VectorSubcoreMesh(core_axis_name="c", subcore_axis_name="s",
                                num_cores=1, num_subcores=NSUB),
    scratch_types=[
        pltpu.VMEM((DATA_WORDS,), jnp.float32),
        pltpu.VMEM((16 * B,), jnp.float32),
        pltpu.VMEM((16 * B,), jnp.float32),
        pltpu.VMEM((BROWS, 16), jnp.float32),
        pltpu.VMEM((BROWS, 16), jnp.float32),
        pltpu.VMEM((16,), jnp.float32),
        pltpu.VMEM((NSUB, 16), jnp.float32),
        pltpu.VMEM((BROWS,), jnp.int32),
        pltpu.VMEM_SHARED((BROWS, 16), jnp.float32),
        pltpu.VMEM_SHARED((BROWS, 16), jnp.float32),
        pltpu.VMEM_SHARED((BROWS, 16), jnp.float32),
        pltpu.VMEM_SHARED((NSUB, 16), jnp.float32),
        pltpu.SemaphoreType.DMA,
    ],
    compiler_params=pltpu.CompilerParams(needs_layout_passes=False,
                                        disable_bounds_checks=True),
)


def kernel(risks):
    out = _cvar_call(risks)
    return out[0]


# speedup vs baseline: 29.2861x; 1.0038x over previous
import jax
import jax.numpy as jnp
from jax import lax
from jax.experimental import pallas as pl
from jax.experimental.pallas import tpu as pltpu
from jax.experimental.pallas import tpu_sc as plsc

N = 1_000_000
K = 500_000
NSUB = 16
CHUNK = 62_496
TAIL = N - NSUB * CHUNK
DATA_WORDS = CHUNK + TAIL
ITERS = CHUNK // 16
ITERS_TAIL = DATA_WORDS // 16
B = 1024
BROWS = B // 16


def _lane0(x):
    return lax.squeeze(lax.slice(x, (0,), (1,)), (0,))


def _find_bucket(ref, lane, target):
    zeros = jnp.zeros((16,), jnp.float32)

    def body(i, carry):
        run, b, cab, cat = carry
        c = (BROWS - 1) - i
        v = ref[c]
        rv = lax.rev(v, (0,))
        rc = lax.cumsum(rv, axis=0)
        s = lax.squeeze(lax.slice(rc, (15,), (16,)), (0,))
        crosses = jnp.logical_and(run < target, run + s >= target)

        def hit_fn():
            hit = (run + rc) >= target
            mv = plsc.all_reduce_ffs(hit)
            sel = lane == mv
            m = _lane0(mv)
            nb = c * 16 + 15 - m
            ncab = run + jnp.sum(jnp.where(sel, rc - rv, zeros))
            ncat = jnp.sum(jnp.where(sel, rv, zeros))
            return (nb, ncab, ncat)

        b, cab, cat = lax.cond(crosses, hit_fn, lambda: (b, cab, cat))
        return (run + s, b, cab, cat)

    f0 = jnp.float32(0)
    _, b, cab, cat = lax.fori_loop(
        0, BROWS, body, (f0, jnp.int32(0), f0, f0))
    return b, cab, cat


def _sortable_key(x):
    u = lax.bitcast_convert_type(x, jnp.int32)
    return u ^ lax.shift_right_logical(lax.shift_right_arithmetic(u, 31), 1)


def _body(risks_hbm, out_hbm, data_v, histc_v, sum2_v, stage1_v, stage2_v,
          stageA_v, stageB_v, idx64_v, merged1_sh, merged2c_sh, merged2s_sh,
          sumab_sh, sem):
    wid = lax.axis_index("s")
    lane = lax.iota(jnp.int32, 16)
    zeros = jnp.zeros((16,), jnp.float32)
    ones = jnp.ones((16,), jnp.float32)
    laneoff = lane * B

    cp = pltpu.async_copy(risks_hbm.at[pl.ds(wid * CHUNK, CHUNK)],
                          data_v.at[pl.ds(0, CHUNK)], sem)

    @plsc.parallel_loop(0, B, unroll=8)
    def _(i):
        histc_v[pl.ds(i * 16, 16)] = zeros

    @plsc.parallel_loop(0, BROWS, unroll=4)
    def _(c):
        stage1_v[c] = zeros
    for c4 in range(BROWS // 16):
        idx64_v[pl.ds(c4 * 16, 16)] = lane + c4 * 16

    @pl.when(wid == 0)
    def _():
        pltpu.sync_copy(stage1_v, merged1_sh)
        pltpu.sync_copy(stage1_v, merged2c_sh)
        pltpu.sync_copy(stage1_v.at[pl.ds(0, 16)], sumab_sh)

    @pl.when(wid == NSUB - 1)
    def _():
        pltpu.sync_copy(risks_hbm.at[pl.ds(NSUB * CHUNK, TAIL)],
                        data_v.at[pl.ds(CHUNK, TAIL)])
    plsc.subcore_barrier()

    cp.wait()
    iters_me = jnp.where(wid == NSUB - 1, ITERS_TAIL, ITERS)

    laneoff512 = laneoff + 512

    @plsc.parallel_loop(0, iters_me, unroll=8)
    def _(i):
        x = data_v[pl.ds(i * 16, 16)]
        key = _sortable_key(x)
        idx = laneoff512 + lax.shift_right_arithmetic(key, 22)
        plsc.addupdate_scatter(histc_v, [idx], ones)

    @plsc.parallel_loop(0, BROWS, unroll=2)
    def _(c):
        acc_ = zeros
        for l in range(16):
            acc_ = acc_ + histc_v[pl.ds(l * B + c * 16, 16)]
        stage1_v[c] = acc_
    pltpu.sync_copy(stage1_v, merged1_sh.at[idx64_v], add=True)

    @plsc.parallel_loop(0, B, unroll=8)
    def _(i):
        histc_v[pl.ds(i * 16, 16)] = zeros
    plsc.subcore_barrier()

    pltpu.sync_copy(merged1_sh, stage1_v)
    kf = jnp.float32(K)
    b1, cab1, _ = _find_bucket(stage1_v, lane, kf)
    rem1 = kf - cab1

    b1s = b1 - 512

    @plsc.parallel_loop(0, iters_me, unroll=8, carry=zeros)
    def acc(i, a):
        x = data_v[pl.ds(i * 16, 16)]
        key = _sortable_key(x)
        b1v = lax.shift_right_arithmetic(key, 22)
        a = a + jnp.where(b1v > b1s, x, zeros)
        inb = b1v == b1s
        idx2 = laneoff + (lax.shift_right_arithmetic(key, 12) & 1023)
        plsc.addupdate_scatter(histc_v, [idx2], ones, mask=inb)
        return a

    @plsc.parallel_loop(0, BROWS, unroll=2)
    def _(c):
        acc1 = zeros
        for l in range(16):
            acc1 = acc1 + histc_v[pl.ds(l * B + c * 16, 16)]
        stage1_v[c] = acc1
    stageA_v[...] = acc
    pltpu.sync_copy(stage1_v, merged2c_sh.at[idx64_v], add=True)
    pltpu.sync_copy(stageA_v, sumab_sh.at[wid])
    plsc.subcore_barrier()

    @pl.when(wid == 0)
    def _():
        pltpu.sync_copy(merged2c_sh, stage1_v)
        pltpu.sync_copy(sumab_sh, stageB_v)
        b2, cab2, _ = _find_bucket(stage1_v, lane, rem1)
        rem2 = rem1 - cab2

        hi_key = lax.shift_left(b1s, 22)

        @plsc.parallel_loop(0, BROWS, unroll=4, carry=(zeros, zeros))
        def fin_acc(c, carry):
            sa_v, ed_v = carry
            vc = stage1_v[c]
            gidx = c * 16 + lane
            ke = hi_key + lax.shift_left(gidx, 12)
            ue = jnp.where(ke < 0, ke ^ jnp.int32(0x7FFFFFFF), ke)
            edge = lax.bitcast_convert_type(ue, jnp.float32)
            sa_v = sa_v + jnp.where(gidx > b2, vc * edge, zeros)
            ed_v = ed_v + jnp.where(gidx == b2, edge, zeros)
            return (sa_v, ed_v)
        s_above2 = jnp.sum(fin_acc[0])
        edge_at = jnp.sum(fin_acc[1])

        @plsc.parallel_loop(0, NSUB, unroll=4, carry=zeros)
        def ts_acc(t, a):
            return a + stageB_v[t]
        sum_above1 = jnp.sum(ts_acc)

        tail = (zeros + rem2) * (zeros + edge_at)
        stageA_v[...] = ((zeros + sum_above1 + s_above2) + tail) / (zeros + kf)
        pltpu.sync_copy(stageA_v, out_hbm)


_cvar_call = pl.kernel(
    _body,
    out_type=jax.ShapeDtypeStruct((16,), jnp.float32),
    mesh=plsc.VectorSubcoreMesh(core_axis_name="c", subcore_axis_name="s",
                                num_cores=1, num_subcores=NSUB),
    scratch_types=[
        pltpu.VMEM((DATA_WORDS,), jnp.float32),
        pltpu.VMEM((16 * B,), jnp.float32),
        pltpu.VMEM((16 * B,), jnp.float32),
        pltpu.VMEM((BROWS, 16), jnp.float32),
        pltpu.VMEM((BROWS, 16), jnp.float32),
        pltpu.VMEM((16,), jnp.float32),
        pltpu.VMEM((NSUB, 16), jnp.float32),
        pltpu.VMEM((BROWS,), jnp.int32),
        pltpu.VMEM_SHARED((BROWS, 16), jnp.float32),
        pltpu.VMEM_SHARED((BROWS, 16), jnp.float32),
        pltpu.VMEM_SHARED((BROWS, 16), jnp.float32),
        pltpu.VMEM_SHARED((NSUB, 16), jnp.float32),
        pltpu.SemaphoreType.DMA,
    ],
    compiler_params=pltpu.CompilerParams(needs_layout_passes=False),
)


def kernel(risks):
    out = _cvar_call(risks)
    return out[0]
